# baseline (speedup 1.0000x reference)
import jax
import jax.numpy as jnp
from jax import lax
from jax.experimental import pallas as pl
from jax.experimental.pallas import tpu as pltpu
from jax.experimental.pallas import tpu_sc as plsc

N = 10000
DF = 128
H = 32
G = 64
E = 320000

NC = 2
NS = 16
NW = NC * NS

CH = 128
CHUNKS = 80
EPW = CH * CHUNKS
EPAD = EPW * NW
CHUNKS1 = EPAD // (NS * CH)
NPAD = 10112
RPS = NPAD // NS
DH = DF // 2

_HIGH = jax.lax.Precision.HIGHEST


def _dot(a, b):
  return jnp.dot(a, b, preferred_element_type=jnp.float32)


def _bn_relu(u, g, b):
  mu = jnp.mean(u, axis=0)
  var = jnp.mean((u - mu) ** 2, axis=0)
  return jnp.maximum((u - mu) * lax.rsqrt(var + 1e-5) * g + b, 0.0)


def _make_sc_agg(W, nchunks, layer1):
  def body(tab_hbm, pidx_hbm, init_hbm, zeros_hbm, out_hbm,
           pk_v, src_c, dst_c, r0v, r1v, r2v, r3v, acc_sh, semg, sems):
    c = lax.axis_index("c")
    s = lax.axis_index("s")
    slab = s if layer1 else s * NC + c
    pltpu.sync_copy(pidx_hbm.at[slab], pk_v)
    row0 = s * RPS
    if layer1:
      pltpu.sync_copy(init_hbm.at[pl.ds(row0, RPS), pl.ds(c * W, W)],
                      acc_sh.at[pl.ds(row0, RPS)])
    else:
      @pl.when(c == 0)
      def _():
        pltpu.sync_copy(init_hbm.at[pl.ds(row0, RPS)],
                        acc_sh.at[pl.ds(row0, RPS)])

      @pl.when(c != 0)
      def _():
        pltpu.sync_copy(zeros_hbm, acc_sh.at[pl.ds(row0, RPS)])
    plsc.subcore_barrier()

    rows = [r0v, r1v, r2v, r3v]

    def unpack(j, t):
      for k in range(CH // 16):
        pk = pk_v[j, pl.ds(k * 16, 16)]
        sv = lax.bitwise_and(pk, 0xFFFF)
        if layer1:
          sv = sv * 2 + c
        src_c[t, pl.ds(k * 16, 16)] = sv
        dst_c[t, pl.ds(k * 16, 16)] = lax.shift_right_logical(pk, 16)

    def gather(j, t):
      unpack(j, t)
      pltpu.async_copy(tab_hbm.at[src_c.at[t]], rows[t], semg.at[t])

    def wait_gather(t):
      pltpu.make_async_copy(tab_hbm.at[src_c.at[t]], rows[t],
                            semg.at[t]).wait()

    def scatter(t):
      pltpu.async_copy(rows[t], acc_sh.at[dst_c.at[t]], sems.at[t], add=True)

    def wait_scatter(t):
      pltpu.make_async_copy(rows[t], acc_sh.at[dst_c.at[t]],
                            sems.at[t]).wait()

    gather(0, 0)
    gather(1, 1)
    nq = nchunks // 4

    @pl.loop(0, nq)
    def _(q):
      j0 = 4 * q

      @pl.when(q > 0)
      def _():
        wait_scatter(2)

      gather(j0 + 2, 2)
      wait_gather(0)
      scatter(0)

      @pl.when(q > 0)
      def _():
        wait_scatter(3)

      gather(j0 + 3, 3)
      wait_gather(1)
      scatter(1)

      @pl.when(q + 1 < nq)
      def _():
        wait_scatter(0)
        gather(j0 + 4, 0)

      wait_gather(2)
      scatter(2)

      @pl.when(q + 1 < nq)
      def _():
        wait_scatter(1)
        gather(j0 + 5, 1)

      wait_gather(3)
      scatter(3)

    for t in range(4):
      wait_scatter(t)
    plsc.subcore_barrier()
    if layer1:
      pltpu.sync_copy(acc_sh.at[pl.ds(row0, RPS)],
                      out_hbm.at[pl.ds(row0, RPS), pl.ds(c * W, W)])
    else:
      pltpu.sync_copy(acc_sh.at[pl.ds(row0, RPS)],
                      out_hbm.at[c].at[pl.ds(row0, RPS)])

  out_sh = (NPAD, DF) if layer1 else (NC, NPAD, W)
  return pl.kernel(
      body,
      out_type=jax.ShapeDtypeStruct(out_sh, jnp.float32),
      mesh=plsc.VectorSubcoreMesh(core_axis_name="c", subcore_axis_name="s"),
      compiler_params=pltpu.CompilerParams(use_tc_tiling_on_sc=False),
      scratch_types=[
          pltpu.VMEM((nchunks, CH), jnp.int32),
          pltpu.VMEM((4, CH), jnp.int32),
          pltpu.VMEM((4, CH), jnp.int32),
          pltpu.VMEM((CH, W), jnp.float32),
          pltpu.VMEM((CH, W), jnp.float32),
          pltpu.VMEM((CH, W), jnp.float32),
          pltpu.VMEM((CH, W), jnp.float32),
          pltpu.VMEM_SHARED((NPAD, W), jnp.float32),
          pltpu.SemaphoreType.DMA((4,)),
          pltpu.SemaphoreType.DMA((4,)),
      ],
  )


_sc_agg1 = _make_sc_agg(DH, CHUNKS1, True)
_sc_agg_h = _make_sc_agg(H, CHUNKS, False)


def _tc_mid1_body(p_ref, wa_ref, ba_ref, ga_ref, bea_ref,
                  wb_ref, bb_ref, gb_ref, beb_ref, xo_ref):
  z = p_ref[:N, :]
  u = _dot(z, wa_ref[...]) + ba_ref[...]
  h1 = _bn_relu(u, ga_ref[...], bea_ref[...])
  u2 = _dot(h1, wb_ref[...]) + bb_ref[...]
  xo_ref[...] = _bn_relu(u2, gb_ref[...], beb_ref[...])


def _tc_mid1(p, wa, ba, ga, bea, wb, bb, gb, beb):
  return pl.pallas_call(
      _tc_mid1_body,
      out_shape=jax.ShapeDtypeStruct((N, H), jnp.float32),
  )(p, wa, ba, ga, bea, wb, bb, gb, beb)


def _tc_mid_body(p_ref, wa_ref, ba_ref, ga_ref, bea_ref,
                 wb_ref, bb_ref, gb_ref, beb_ref, xo_ref):
  z = p_ref[0, :N, :] + p_ref[1, :N, :]
  u = _dot(z, wa_ref[...]) + ba_ref[...]
  h1 = _bn_relu(u, ga_ref[...], bea_ref[...])
  u2 = _dot(h1, wb_ref[...]) + bb_ref[...]
  xo_ref[...] = _bn_relu(u2, gb_ref[...], beb_ref[...])


def _tc_mid(p, wa, ba, ga, bea, wb, bb, gb, beb):
  return pl.pallas_call(
      _tc_mid_body,
      out_shape=jax.ShapeDtypeStruct((N, H), jnp.float32),
  )(p, wa, ba, ga, bea, wb, bb, gb, beb)


def _tc_final_body(p_ref, wa_ref, ba_ref, ga_ref,
                   bea_ref, wb_ref, bb_ref, gb_ref, beb_ref, x1_ref, x2_ref,
                   wlin_ref, blin_ref, wg_ref, bg_ref, batch_ref,
                   hn_ref, pool_ref):
  z = p_ref[0, :N, :] + p_ref[1, :N, :]
  u = _dot(z, wa_ref[...]) + ba_ref[...]
  h1 = _bn_relu(u, ga_ref[...], bea_ref[...])
  u2 = _dot(h1, wb_ref[...]) + bb_ref[...]
  x3 = _bn_relu(u2, gb_ref[...], beb_ref[...])

  hcat = jnp.concatenate([x1_ref[...], x2_ref[...], x3], axis=1)
  hnode = jnp.maximum(_dot(hcat, wlin_ref[...]) + blin_ref[...], 0.0)
  hn_ref[...] = hnode

  gate = _dot(hnode, wg_ref[...]) + bg_ref[...]
  seg = lax.broadcasted_iota(jnp.int32, (N, G), 1)
  mask = seg == batch_ref[...]
  maskf = mask.astype(jnp.float32)
  neg = jnp.float32(-jnp.inf)
  gmax = jnp.max(jnp.where(mask, gate, neg), axis=0)
  gmax_n = jnp.max(jnp.where(mask, gmax[None, :], neg), axis=1, keepdims=True)
  ex = jnp.exp(gate - gmax_n)
  den = jnp.sum(jnp.where(mask, ex, 0.0), axis=0)
  den_n = jnp.sum(jnp.where(mask, den[None, :], 0.0), axis=1, keepdims=True)
  alpha = ex / (den_n + 1e-16)
  pool_ref[...] = lax.dot_general(
      maskf, alpha * hnode, (((0,), (0,)), ((), ())),
      precision=_HIGH, preferred_element_type=jnp.float32)


def _tc_final(p, wa, ba, ga, bea, wb, bb, gb, beb, x1, x2,
              wlin, blin, wg, bg, batch2):
  return pl.pallas_call(
      _tc_final_body,
      out_shape=[
          jax.ShapeDtypeStruct((N, H), jnp.float32),
          jax.ShapeDtypeStruct((G, H), jnp.float32),
      ],
  )(p, wa, ba, ga, bea, wb, bb, gb, beb, x1, x2,
    wlin, blin, wg, bg, batch2)


def kernel(x, edge_index, batch,
           eps1, W1a, b1a, g1a, be1a, W1b, b1b, g1b, be1b,
           eps2, W2a, b2a, g2a, be2a, W2b, b2b, g2b, be2b,
           eps3, W3a, b3a, g3a, be3a, W3b, b3b, g3b, be3b,
           Wlin, blin, Wg, bg):
  src = edge_index[0]
  dst = edge_index[1]
  pad = EPAD - E
  pidx = jnp.arange(pad, dtype=jnp.int32)
  srcp = jnp.concatenate([src, pidx % N])
  dstp = jnp.concatenate([dst, N + pidx % (NPAD - N)])
  pidx = srcp | (dstp << 16)
  pidx1 = pidx.reshape(NS, CHUNKS1, CH)
  pidx2 = pidx.reshape(NW, CHUNKS, CH)
  x2v = x.reshape(2 * N, DH)
  zeros_dh = jnp.zeros((RPS, DH), jnp.float32)
  zeros_h = jnp.zeros((RPS, H), jnp.float32)
  batch2 = batch.reshape(N, 1)

  def pad_scale(h, eps):
    return jnp.zeros((NPAD, h.shape[1]), jnp.float32).at[:N].set(
        (1.0 + eps) * h)

  p1 = _sc_agg1(x2v, pidx1, pad_scale(x, eps1), zeros_dh)
  x1 = _tc_mid1(p1, W1a, b1a, g1a, be1a, W1b, b1b, g1b, be1b)
  p2 = _sc_agg_h(x1, pidx2, pad_scale(x1, eps2), zeros_h)
  x2 = _tc_mid(p2, W2a, b2a, g2a, be2a, W2b, b2b, g2b, be2b)
  p3 = _sc_agg_h(x2, pidx2, pad_scale(x2, eps3), zeros_h)
  hnode, pooled = _tc_final(p3, W3a, b3a, g3a, be3a,
                            W3b, b3b, g3b, be3b, x1, x2,
                            Wlin, blin, Wg, bg, batch2)
  return (hnode, pooled)


# speedup vs baseline: 1.0352x; 1.0352x over previous
import jax
import jax.numpy as jnp
from jax import lax
from jax.experimental import pallas as pl
from jax.experimental.pallas import tpu as pltpu
from jax.experimental.pallas import tpu_sc as plsc

N = 10000
DF = 128
H = 32
G = 64
E = 320000

NC = 2
NS = 16
NW = NC * NS

CH = 128
CHUNKS = 80
EPW = CH * CHUNKS
EPAD = EPW * NW
CHUNKS1 = EPAD // (NS * CH)
NPAD = 10112
RPS = NPAD // NS
DH = DF // 2

_HIGH = jax.lax.Precision.HIGHEST


def _dot(a, b):
  return jnp.dot(a, b, preferred_element_type=jnp.float32)


def _bn_relu(u, g, b):
  mu = jnp.mean(u, axis=0)
  var = jnp.mean((u - mu) ** 2, axis=0)
  return jnp.maximum((u - mu) * lax.rsqrt(var + 1e-5) * g + b, 0.0)


def _make_sc_agg(W, nchunks, layer1):
  def body(tab_hbm, pidx_hbm, zeros_hbm, out_hbm,
           pk_v, src_c, dst_c, r0v, r1v, r2v, r3v, acc_sh, semg, sems):
    c = lax.axis_index("c")
    s = lax.axis_index("s")
    slab = s if layer1 else s * NC + c
    pltpu.sync_copy(pidx_hbm.at[slab], pk_v)
    row0 = s * RPS
    pltpu.sync_copy(zeros_hbm, acc_sh.at[pl.ds(row0, RPS)])
    plsc.subcore_barrier()

    rows = [r0v, r1v, r2v, r3v]

    def unpack(j, t):
      for k in range(CH // 16):
        pk = pk_v[j, pl.ds(k * 16, 16)]
        sv = lax.bitwise_and(pk, 0xFFFF)
        if layer1:
          sv = sv * 2 + c
        src_c[t, pl.ds(k * 16, 16)] = sv
        dst_c[t, pl.ds(k * 16, 16)] = lax.shift_right_logical(pk, 16)

    def gather(j, t):
      unpack(j, t)
      pltpu.async_copy(tab_hbm.at[src_c.at[t]], rows[t], semg.at[t])

    def wait_gather(t):
      pltpu.make_async_copy(tab_hbm.at[src_c.at[t]], rows[t],
                            semg.at[t]).wait()

    def scatter(t):
      pltpu.async_copy(rows[t], acc_sh.at[dst_c.at[t]], sems.at[t], add=True)

    def wait_scatter(t):
      pltpu.make_async_copy(rows[t], acc_sh.at[dst_c.at[t]],
                            sems.at[t]).wait()

    gather(0, 0)
    gather(1, 1)
    nq = nchunks // 4

    @pl.loop(0, nq)
    def _(q):
      j0 = 4 * q

      @pl.when(q > 0)
      def _():
        wait_scatter(2)

      gather(j0 + 2, 2)
      wait_gather(0)
      scatter(0)

      @pl.when(q > 0)
      def _():
        wait_scatter(3)

      gather(j0 + 3, 3)
      wait_gather(1)
      scatter(1)

      @pl.when(q + 1 < nq)
      def _():
        wait_scatter(0)
        gather(j0 + 4, 0)

      wait_gather(2)
      scatter(2)

      @pl.when(q + 1 < nq)
      def _():
        wait_scatter(1)
        gather(j0 + 5, 1)

      wait_gather(3)
      scatter(3)

    for t in range(4):
      wait_scatter(t)
    plsc.subcore_barrier()
    if layer1:
      pltpu.sync_copy(acc_sh.at[pl.ds(row0, RPS)],
                      out_hbm.at[pl.ds(row0, RPS), pl.ds(c * W, W)])
    else:
      pltpu.sync_copy(acc_sh.at[pl.ds(row0, RPS)],
                      out_hbm.at[c].at[pl.ds(row0, RPS)])

  out_sh = (NPAD, DF) if layer1 else (NC, NPAD, W)
  return pl.kernel(
      body,
      out_type=jax.ShapeDtypeStruct(out_sh, jnp.float32),
      mesh=plsc.VectorSubcoreMesh(core_axis_name="c", subcore_axis_name="s"),
      compiler_params=pltpu.CompilerParams(use_tc_tiling_on_sc=False),
      scratch_types=[
          pltpu.VMEM((nchunks, CH), jnp.int32),
          pltpu.VMEM((4, CH), jnp.int32),
          pltpu.VMEM((4, CH), jnp.int32),
          pltpu.VMEM((CH, W), jnp.float32),
          pltpu.VMEM((CH, W), jnp.float32),
          pltpu.VMEM((CH, W), jnp.float32),
          pltpu.VMEM((CH, W), jnp.float32),
          pltpu.VMEM_SHARED((NPAD, W), jnp.float32),
          pltpu.SemaphoreType.DMA((4,)),
          pltpu.SemaphoreType.DMA((4,)),
      ],
  )


_sc_agg1 = _make_sc_agg(DH, CHUNKS1, True)
_sc_agg_h = _make_sc_agg(H, CHUNKS, False)


def _tc_mid1_body(h_ref, p_ref, eps_ref, wa_ref, ba_ref, ga_ref, bea_ref,
                  wb_ref, bb_ref, gb_ref, beb_ref, xo_ref):
  z = (1.0 + eps_ref[0, 0]) * h_ref[...] + p_ref[:N, :]
  u = _dot(z, wa_ref[...]) + ba_ref[...]
  h1 = _bn_relu(u, ga_ref[...], bea_ref[...])
  u2 = _dot(h1, wb_ref[...]) + bb_ref[...]
  xo_ref[...] = _bn_relu(u2, gb_ref[...], beb_ref[...])


def _tc_mid1(h, p, eps, wa, ba, ga, bea, wb, bb, gb, beb):
  return pl.pallas_call(
      _tc_mid1_body,
      out_shape=jax.ShapeDtypeStruct((N, H), jnp.float32),
  )(h, p, eps, wa, ba, ga, bea, wb, bb, gb, beb)


def _tc_mid_body(h_ref, p_ref, eps_ref, wa_ref, ba_ref, ga_ref, bea_ref,
                 wb_ref, bb_ref, gb_ref, beb_ref, xo_ref):
  z = (1.0 + eps_ref[0, 0]) * h_ref[...] + (p_ref[0, :N, :] + p_ref[1, :N, :])
  u = _dot(z, wa_ref[...]) + ba_ref[...]
  h1 = _bn_relu(u, ga_ref[...], bea_ref[...])
  u2 = _dot(h1, wb_ref[...]) + bb_ref[...]
  xo_ref[...] = _bn_relu(u2, gb_ref[...], beb_ref[...])


def _tc_mid(h, p, eps, wa, ba, ga, bea, wb, bb, gb, beb):
  return pl.pallas_call(
      _tc_mid_body,
      out_shape=jax.ShapeDtypeStruct((N, H), jnp.float32),
  )(h, p, eps, wa, ba, ga, bea, wb, bb, gb, beb)


def _tc_final_body(p_ref, eps_ref, wa_ref, ba_ref, ga_ref,
                   bea_ref, wb_ref, bb_ref, gb_ref, beb_ref, x1_ref, x2_ref,
                   wlin_ref, blin_ref, wg_ref, bg_ref, batch_ref,
                   hn_ref, pool_ref):
  z = ((1.0 + eps_ref[0, 0]) * x2_ref[...]
       + (p_ref[0, :N, :] + p_ref[1, :N, :]))
  u = _dot(z, wa_ref[...]) + ba_ref[...]
  h1 = _bn_relu(u, ga_ref[...], bea_ref[...])
  u2 = _dot(h1, wb_ref[...]) + bb_ref[...]
  x3 = _bn_relu(u2, gb_ref[...], beb_ref[...])

  hcat = jnp.concatenate([x1_ref[...], x2_ref[...], x3], axis=1)
  hnode = jnp.maximum(_dot(hcat, wlin_ref[...]) + blin_ref[...], 0.0)
  hn_ref[...] = hnode

  gate = _dot(hnode, wg_ref[...]) + bg_ref[...]
  seg = lax.broadcasted_iota(jnp.int32, (N, G), 1)
  mask = seg == batch_ref[...]
  maskf = mask.astype(jnp.float32)
  neg = jnp.float32(-jnp.inf)
  gmax = jnp.max(jnp.where(mask, gate, neg), axis=0)
  gmax_n = jnp.max(jnp.where(mask, gmax[None, :], neg), axis=1, keepdims=True)
  ex = jnp.exp(gate - gmax_n)
  den = jnp.sum(jnp.where(mask, ex, 0.0), axis=0)
  den_n = jnp.sum(jnp.where(mask, den[None, :], 0.0), axis=1, keepdims=True)
  alpha = ex / (den_n + 1e-16)
  pool_ref[...] = lax.dot_general(
      maskf, alpha * hnode, (((0,), (0,)), ((), ())),
      precision=_HIGH, preferred_element_type=jnp.float32)


def _tc_final(p, eps, wa, ba, ga, bea, wb, bb, gb, beb, x1, x2,
              wlin, blin, wg, bg, batch2):
  return pl.pallas_call(
      _tc_final_body,
      out_shape=[
          jax.ShapeDtypeStruct((N, H), jnp.float32),
          jax.ShapeDtypeStruct((G, H), jnp.float32),
      ],
  )(p, eps, wa, ba, ga, bea, wb, bb, gb, beb, x1, x2,
    wlin, blin, wg, bg, batch2)


def kernel(x, edge_index, batch,
           eps1, W1a, b1a, g1a, be1a, W1b, b1b, g1b, be1b,
           eps2, W2a, b2a, g2a, be2a, W2b, b2b, g2b, be2b,
           eps3, W3a, b3a, g3a, be3a, W3b, b3b, g3b, be3b,
           Wlin, blin, Wg, bg):
  src = edge_index[0]
  dst = edge_index[1]
  pad = EPAD - E
  pidx = jnp.arange(pad, dtype=jnp.int32)
  srcp = jnp.concatenate([src, pidx % N])
  dstp = jnp.concatenate([dst, N + pidx % (NPAD - N)])
  pidx = srcp | (dstp << 16)
  pidx1 = pidx.reshape(NS, CHUNKS1, CH)
  pidx2 = pidx.reshape(NW, CHUNKS, CH)
  x2v = x.reshape(2 * N, DH)
  zeros_dh = jnp.zeros((RPS, DH), jnp.float32)
  zeros_h = jnp.zeros((RPS, H), jnp.float32)
  batch2 = batch.reshape(N, 1)
  e1 = eps1.reshape(1, 1)
  e2 = eps2.reshape(1, 1)
  e3 = eps3.reshape(1, 1)

  p1 = _sc_agg1(x2v, pidx1, zeros_dh)
  x1 = _tc_mid1(x, p1, e1, W1a, b1a, g1a, be1a, W1b, b1b, g1b, be1b)
  p2 = _sc_agg_h(x1, pidx2, zeros_h)
  x2 = _tc_mid(x1, p2, e2, W2a, b2a, g2a, be2a, W2b, b2b, g2b, be2b)
  p3 = _sc_agg_h(x2, pidx2, zeros_h)
  hnode, pooled = _tc_final(p3, e3, W3a, b3a, g3a, be3a,
                            W3b, b3b, g3b, be3b, x1, x2,
                            Wlin, blin, Wg, bg, batch2)
  return (hnode, pooled)


# speedup vs baseline: 1.0421x; 1.0067x over previous
import jax
import jax.numpy as jnp
from jax import lax
from jax.experimental import pallas as pl
from jax.experimental.pallas import tpu as pltpu
from jax.experimental.pallas import tpu_sc as plsc

N = 10000
DF = 128
H = 32
G = 64
E = 320000

NC = 2
NS = 16
NW = NC * NS

CH = 128
CHUNKS = 80
EPW = CH * CHUNKS
EPAD = EPW * NW
CHUNKS1 = EPAD // (NS * CH)
NPAD = 10112
RPS = NPAD // NS
DH = DF // 2

_HIGH = jax.lax.Precision.HIGHEST


def _dot(a, b):
  return jnp.dot(a, b, preferred_element_type=jnp.float32)


def _bn_relu(u, g, b):
  mu = jnp.mean(u, axis=0)
  var = jnp.mean((u - mu) ** 2, axis=0)
  return jnp.maximum((u - mu) * lax.rsqrt(var + 1e-5) * g + b, 0.0)


def _make_sc_agg(W, nchunks, layer1):
  def body(tab_hbm, pidx_hbm, zeros_hbm, out_hbm,
           pk_v, src_c, dst_c, r0v, r1v, r2v, r3v, acc_sh, semg, sems):
    c = lax.axis_index("c")
    s = lax.axis_index("s")
    slab = s if layer1 else s * NC + c
    row0 = s * RPS
    pltpu.async_copy(pidx_hbm.at[slab], pk_v, semg.at[0])
    pltpu.async_copy(zeros_hbm, acc_sh.at[pl.ds(row0, RPS)], semg.at[1])
    pltpu.make_async_copy(pidx_hbm.at[slab], pk_v, semg.at[0]).wait()
    pltpu.make_async_copy(zeros_hbm, acc_sh.at[pl.ds(row0, RPS)],
                          semg.at[1]).wait()
    plsc.subcore_barrier()

    rows = [r0v, r1v, r2v, r3v]

    def unpack(j, t):
      for k in range(CH // 16):
        pk = pk_v[j, pl.ds(k * 16, 16)]
        sv = lax.bitwise_and(pk, 0xFFFF)
        if layer1:
          sv = sv * 2 + c
        src_c[t, pl.ds(k * 16, 16)] = sv
        dst_c[t, pl.ds(k * 16, 16)] = lax.shift_right_logical(pk, 16)

    def gather(j, t):
      unpack(j, t)
      pltpu.async_copy(tab_hbm.at[src_c.at[t]], rows[t], semg.at[t])

    def wait_gather(t):
      pltpu.make_async_copy(tab_hbm.at[src_c.at[t]], rows[t],
                            semg.at[t]).wait()

    def scatter(t):
      pltpu.async_copy(rows[t], acc_sh.at[dst_c.at[t]], sems.at[t], add=True)

    def wait_scatter(t):
      pltpu.make_async_copy(rows[t], acc_sh.at[dst_c.at[t]],
                            sems.at[t]).wait()

    gather(0, 0)
    gather(1, 1)
    nq = nchunks // 4

    @pl.loop(0, nq)
    def _(q):
      j0 = 4 * q

      @pl.when(q > 0)
      def _():
        wait_scatter(2)

      gather(j0 + 2, 2)
      wait_gather(0)
      scatter(0)

      @pl.when(q > 0)
      def _():
        wait_scatter(3)

      gather(j0 + 3, 3)
      wait_gather(1)
      scatter(1)

      @pl.when(q + 1 < nq)
      def _():
        wait_scatter(0)
        gather(j0 + 4, 0)

      wait_gather(2)
      scatter(2)

      @pl.when(q + 1 < nq)
      def _():
        wait_scatter(1)
        gather(j0 + 5, 1)

      wait_gather(3)
      scatter(3)

    for t in range(4):
      wait_scatter(t)
    plsc.subcore_barrier()
    if layer1:
      pltpu.sync_copy(acc_sh.at[pl.ds(row0, RPS)],
                      out_hbm.at[pl.ds(row0, RPS), pl.ds(c * W, W)])
    else:
      pltpu.sync_copy(acc_sh.at[pl.ds(row0, RPS)],
                      out_hbm.at[c].at[pl.ds(row0, RPS)])

  out_sh = (NPAD, DF) if layer1 else (NC, NPAD, W)
  return pl.kernel(
      body,
      out_type=jax.ShapeDtypeStruct(out_sh, jnp.float32),
      mesh=plsc.VectorSubcoreMesh(core_axis_name="c", subcore_axis_name="s"),
      compiler_params=pltpu.CompilerParams(use_tc_tiling_on_sc=False),
      scratch_types=[
          pltpu.VMEM((nchunks, CH), jnp.int32),
          pltpu.VMEM((4, CH), jnp.int32),
          pltpu.VMEM((4, CH), jnp.int32),
          pltpu.VMEM((CH, W), jnp.float32),
          pltpu.VMEM((CH, W), jnp.float32),
          pltpu.VMEM((CH, W), jnp.float32),
          pltpu.VMEM((CH, W), jnp.float32),
          pltpu.VMEM_SHARED((NPAD, W), jnp.float32),
          pltpu.SemaphoreType.DMA((4,)),
          pltpu.SemaphoreType.DMA((4,)),
      ],
  )


_sc_agg1 = _make_sc_agg(DH, CHUNKS1, True)
_sc_agg_h = _make_sc_agg(H, CHUNKS, False)


def _tc_mid1_body(h_ref, p_ref, eps_ref, wa_ref, ba_ref, ga_ref, bea_ref,
                  wb_ref, bb_ref, gb_ref, beb_ref, xo_ref):
  z = (1.0 + eps_ref[0, 0]) * h_ref[...] + p_ref[:N, :]
  u = _dot(z, wa_ref[...]) + ba_ref[...]
  h1 = _bn_relu(u, ga_ref[...], bea_ref[...])
  u2 = _dot(h1, wb_ref[...]) + bb_ref[...]
  xo_ref[...] = _bn_relu(u2, gb_ref[...], beb_ref[...])


def _tc_mid1(h, p, eps, wa, ba, ga, bea, wb, bb, gb, beb):
  return pl.pallas_call(
      _tc_mid1_body,
      out_shape=jax.ShapeDtypeStruct((N, H), jnp.float32),
  )(h, p, eps, wa, ba, ga, bea, wb, bb, gb, beb)


def _tc_mid_body(h_ref, p_ref, eps_ref, wa_ref, ba_ref, ga_ref, bea_ref,
                 wb_ref, bb_ref, gb_ref, beb_ref, xo_ref):
  z = (1.0 + eps_ref[0, 0]) * h_ref[...] + (p_ref[0, :N, :] + p_ref[1, :N, :])
  u = _dot(z, wa_ref[...]) + ba_ref[...]
  h1 = _bn_relu(u, ga_ref[...], bea_ref[...])
  u2 = _dot(h1, wb_ref[...]) + bb_ref[...]
  xo_ref[...] = _bn_relu(u2, gb_ref[...], beb_ref[...])


def _tc_mid(h, p, eps, wa, ba, ga, bea, wb, bb, gb, beb):
  return pl.pallas_call(
      _tc_mid_body,
      out_shape=jax.ShapeDtypeStruct((N, H), jnp.float32),
  )(h, p, eps, wa, ba, ga, bea, wb, bb, gb, beb)


def _tc_final_body(p_ref, eps_ref, wa_ref, ba_ref, ga_ref,
                   bea_ref, wb_ref, bb_ref, gb_ref, beb_ref, x1_ref, x2_ref,
                   wlin_ref, blin_ref, wg_ref, bg_ref, batch_ref,
                   hn_ref, pool_ref):
  z = ((1.0 + eps_ref[0, 0]) * x2_ref[...]
       + (p_ref[0, :N, :] + p_ref[1, :N, :]))
  u = _dot(z, wa_ref[...]) + ba_ref[...]
  h1 = _bn_relu(u, ga_ref[...], bea_ref[...])
  u2 = _dot(h1, wb_ref[...]) + bb_ref[...]
  x3 = _bn_relu(u2, gb_ref[...], beb_ref[...])

  hcat = jnp.concatenate([x1_ref[...], x2_ref[...], x3], axis=1)
  hnode = jnp.maximum(_dot(hcat, wlin_ref[...]) + blin_ref[...], 0.0)
  hn_ref[...] = hnode

  gate = _dot(hnode, wg_ref[...]) + bg_ref[...]
  seg = lax.broadcasted_iota(jnp.int32, (N, G), 1)
  mask = seg == batch_ref[...]
  maskf = mask.astype(jnp.float32)
  neg = jnp.float32(-jnp.inf)
  gmax = jnp.max(jnp.where(mask, gate, neg), axis=0)
  gmax_n = jnp.max(jnp.where(mask, gmax[None, :], neg), axis=1, keepdims=True)
  ex = jnp.exp(gate - gmax_n)
  den = jnp.sum(jnp.where(mask, ex, 0.0), axis=0)
  den_n = jnp.sum(jnp.where(mask, den[None, :], 0.0), axis=1, keepdims=True)
  alpha = ex / (den_n + 1e-16)
  pool_ref[...] = lax.dot_general(
      maskf, alpha * hnode, (((0,), (0,)), ((), ())),
      precision=_HIGH, preferred_element_type=jnp.float32)


def _tc_final(p, eps, wa, ba, ga, bea, wb, bb, gb, beb, x1, x2,
              wlin, blin, wg, bg, batch2):
  return pl.pallas_call(
      _tc_final_body,
      out_shape=[
          jax.ShapeDtypeStruct((N, H), jnp.float32),
          jax.ShapeDtypeStruct((G, H), jnp.float32),
      ],
  )(p, eps, wa, ba, ga, bea, wb, bb, gb, beb, x1, x2,
    wlin, blin, wg, bg, batch2)


def kernel(x, edge_index, batch,
           eps1, W1a, b1a, g1a, be1a, W1b, b1b, g1b, be1b,
           eps2, W2a, b2a, g2a, be2a, W2b, b2b, g2b, be2b,
           eps3, W3a, b3a, g3a, be3a, W3b, b3b, g3b, be3b,
           Wlin, blin, Wg, bg):
  src = edge_index[0]
  dst = edge_index[1]
  pad = EPAD - E
  pidx = jnp.arange(pad, dtype=jnp.int32)
  srcp = jnp.concatenate([src, pidx % N])
  dstp = jnp.concatenate([dst, N + pidx % (NPAD - N)])
  pidx = srcp | (dstp << 16)
  pidx1 = pidx.reshape(NS, CHUNKS1, CH)
  pidx2 = pidx.reshape(NW, CHUNKS, CH)
  x2v = x.reshape(2 * N, DH)
  zeros_dh = jnp.zeros((RPS, DH), jnp.float32)
  zeros_h = jnp.zeros((RPS, H), jnp.float32)
  batch2 = batch.reshape(N, 1)
  e1 = eps1.reshape(1, 1)
  e2 = eps2.reshape(1, 1)
  e3 = eps3.reshape(1, 1)

  p1 = _sc_agg1(x2v, pidx1, zeros_dh)
  x1 = _tc_mid1(x, p1, e1, W1a, b1a, g1a, be1a, W1b, b1b, g1b, be1b)
  p2 = _sc_agg_h(x1, pidx2, zeros_h)
  x2 = _tc_mid(x1, p2, e2, W2a, b2a, g2a, be2a, W2b, b2b, g2b, be2b)
  p3 = _sc_agg_h(x2, pidx2, zeros_h)
  hnode, pooled = _tc_final(p3, e3, W3a, b3a, g3a, be3a,
                            W3b, b3b, g3b, be3b, x1, x2,
                            Wlin, blin, Wg, bg, batch2)
  return (hnode, pooled)
